# baseline (speedup 1.0000x reference)
import functools

import jax
import jax.numpy as jnp
from jax import lax
from jax.experimental import pallas as pl
from jax.experimental.pallas import tpu as pltpu
from jax.experimental.pallas import tpu_sc as plsc

_NC = 2
_NS = 16
_IW = 128
_IB = 16
_SHARE0 = 0.96


def _make_edge_pass(n, f, nrow, rpt0, rpt1, dtype=jnp.float32):
    osl = -(-n // (_NS * 8)) * 8
    n_pad = max(osl * _NS, n + 8)
    zsl = n_pad // _NS
    last = n - (_NS - 1) * osl
    assert last > 0 and n_pad % _NS == 0 and zsl % 8 == 0
    mesh = plsc.VectorSubcoreMesh(
        core_axis_name="c", subcore_axis_name="s",
        num_cores=_NC, num_subcores=_NS)

    assert rpt0 % _IB == 0 and rpt1 % _IB == 0
    k_deep = 4 if (f > 8 and dtype == jnp.float32) else 8

    @functools.partial(
        pl.kernel,
        out_type=jax.ShapeDtypeStruct((_NC, n, f), dtype),
        mesh=mesh,
        compiler_params=pltpu.CompilerParams(use_tc_tiling_on_sc=False),
        scratch_types=[
            pltpu.VMEM((2, _IB, _IW), jnp.int32),
            pltpu.VMEM((2, _IB, _IW), jnp.int32),
            pltpu.VMEM((k_deep, _IW, f), dtype),
            pltpu.VMEM_SHARED((n_pad, f), dtype),
        ] + [pltpu.SemaphoreType.DMA] * (k_deep + 3),
    )
    def edge_pass(src_hbm, dst_hbm, tbl_hbm, zeros_hbm, out_hbm,
                  sblk, dblk, gbuf, acc, *sems):
        isems, gsems, ssem = sems[:2], sems[2:2 + k_deep], sems[2 + k_deep]
        cid = lax.axis_index("c")
        sid = lax.axis_index("s")

        pltpu.sync_copy(zeros_hbm, acc.at[pl.ds(sid * zsl, zsl)])
        plsc.subcore_barrier()

        def fire_idx(base, blk, slot):
            off = base + blk * _IB
            return (
                pltpu.async_copy(src_hbm.at[pl.ds(off, _IB)],
                                 sblk.at[slot, pl.ds(0, _IB)], isems[slot]),
                pltpu.async_copy(dst_hbm.at[pl.ds(off, _IB)],
                                 dblk.at[slot, pl.ds(0, _IB)], isems[slot]),
            )

        def run_rows(rpt_c, base):
            nblk = rpt_c // _IB
            if nblk == 0:
                return
            idx_d = fire_idx(base, 0, 0)
            for blk in range(nblk):
                slot = blk % 2
                idx_d[0].wait()
                idx_d[1].wait()
                if blk + 1 < nblk:
                    idx_d = fire_idx(base, blk + 1, 1 - slot)

                def group(g, carry, slot=slot):
                    g0 = g * k_deep
                    gds = [
                        pltpu.async_copy(tbl_hbm.at[sblk.at[slot, g0 + k]],
                                         gbuf.at[k], gsems[k])
                        for k in range(k_deep)
                    ]
                    sds = []
                    for k in range(k_deep):
                        gds[k].wait()
                        sds.append(
                            pltpu.async_copy(gbuf.at[k],
                                             acc.at[dblk.at[slot, g0 + k]],
                                             ssem, add=True))
                    for sd in sds:
                        sd.wait()
                    return carry

                lax.fori_loop(0, _IB // k_deep, group, 0)

        @pl.when(cid == 0)
        def _core0():
            run_rows(rpt0, sid * rpt0)

        @pl.when(cid == 1)
        def _core1():
            run_rows(rpt1, _NS * rpt0 + sid * rpt1)

        plsc.subcore_barrier()

        @pl.when(sid < _NS - 1)
        def _copy_main():
            pltpu.sync_copy(acc.at[pl.ds(sid * osl, osl)],
                            out_hbm.at[cid, pl.ds(sid * osl, osl)])

        @pl.when(sid == _NS - 1)
        def _copy_tail():
            pltpu.sync_copy(acc.at[pl.ds((_NS - 1) * osl, last)],
                            out_hbm.at[cid, pl.ds((_NS - 1) * osl, last)])

    return edge_pass


def _small_matmul(h, w_ref):
    f = w_ref.shape[0]
    acc = h[:, 0:1] * w_ref[0:1, :]
    for k in range(1, f):
        acc = acc + h[:, k:k + 1] * w_ref[k:k + 1, :]
    return acc


def _tc_pre(n, bsz, fin, fout):
    def body(degp_ref, x_ref, w_ref, dinv_ref, g_ref):
        deg = degp_ref[0] + degp_ref[1] + 1.0
        dinv = lax.rsqrt(deg)
        dinv_ref[...] = dinv
        g_ref[...] = (jnp.dot(x_ref[...], w_ref[...],
                              preferred_element_type=jnp.float32)
                      * dinv).astype(g_ref.dtype)

    return pl.pallas_call(
        body,
        grid=(n // bsz,),
        in_specs=[pl.BlockSpec((2, bsz, 1), lambda i: (0, i, 0)),
                  pl.BlockSpec((bsz, fin), lambda i: (i, 0)),
                  pl.BlockSpec((fin, fout), lambda i: (0, 0))],
        out_specs=[pl.BlockSpec((bsz, 1), lambda i: (i, 0)),
                   pl.BlockSpec((bsz, fout), lambda i: (i, 0))],
        out_shape=[jax.ShapeDtypeStruct((n, 1), jnp.float32),
                   jax.ShapeDtypeStruct((n, fout), jnp.bfloat16)])


def _tc_mid(n, bsz, f, f2, want_h):
    def body(p_ref, g_ref, dinv_ref, b_ref, w_ref, *out_refs):
        dinv = dinv_ref[...]
        s = (p_ref[0].astype(jnp.float32) + p_ref[1].astype(jnp.float32)
             + g_ref[...].astype(jnp.float32)) * dinv + b_ref[...]
        h = jnp.tanh(s)
        if want_h:
            out_refs[0][...] = h
        gn = _small_matmul(h, w_ref) if f <= 8 else jnp.dot(
            h, w_ref[...], preferred_element_type=jnp.float32)
        out_refs[-1][...] = gn * dinv

    out_specs = [pl.BlockSpec((bsz, f2), lambda i: (i, 0))]
    out_shape = [jax.ShapeDtypeStruct((n, f2), jnp.float32)]
    if want_h:
        out_specs.insert(0, pl.BlockSpec((bsz, f), lambda i: (i, 0)))
        out_shape.insert(0, jax.ShapeDtypeStruct((n, f), jnp.float32))
    return pl.pallas_call(
        body,
        grid=(n // bsz,),
        in_specs=[pl.BlockSpec((2, bsz, f), lambda i: (0, i, 0)),
                  pl.BlockSpec((bsz, f), lambda i: (i, 0)),
                  pl.BlockSpec((bsz, 1), lambda i: (i, 0)),
                  pl.BlockSpec((1, f), lambda i: (0, 0)),
                  pl.BlockSpec((f, f2), lambda i: (0, 0))],
        out_specs=out_specs,
        out_shape=out_shape)


def _tc_post(n, bsz, f):
    def body(p_ref, g_ref, dinv_ref, b_ref, y_ref):
        zp = (p_ref[0] + p_ref[1] + g_ref[...]) * dinv_ref[...]
        z = zp[:, :f] + b_ref[...]
        zm = z - jnp.max(z, axis=1, keepdims=True)
        y_ref[...] = zm - jnp.log(jnp.sum(jnp.exp(zm), axis=1, keepdims=True))

    fp = max(f, 8)
    return pl.pallas_call(
        body,
        grid=(n // bsz,),
        in_specs=[pl.BlockSpec((2, bsz, fp), lambda i: (0, i, 0)),
                  pl.BlockSpec((bsz, fp), lambda i: (i, 0)),
                  pl.BlockSpec((bsz, 1), lambda i: (i, 0)),
                  pl.BlockSpec((1, f), lambda i: (0, 0))],
        out_specs=pl.BlockSpec((bsz, f), lambda i: (i, 0)),
        out_shape=jax.ShapeDtypeStruct((n, f), jnp.float32))


def kernel(x, edge_index, W1, b1, W2, b2, W3, b3, W4, b4):
    n = x.shape[0]
    e = edge_index.shape[1]
    nw = _NC * _NS
    e_pad = -(-e // (nw * _IW * 8)) * (nw * _IW * 8)
    nrow = e_pad // _IW
    rpt_pair = nrow // _NS
    rpt0 = min(rpt_pair, max(_IB, int(round(rpt_pair * _SHARE0 / _IB)) * _IB))
    rpt1 = rpt_pair - rpt0
    osl = -(-n // (_NS * 8)) * 8
    zsl = max(osl * _NS, n + 8) // _NS
    bsz = 2000 if n % 2000 == 0 else n // 16

    src = edge_index[0].astype(jnp.int32)
    dst = edge_index[1].astype(jnp.int32)
    pad = e_pad - e
    src2d = jnp.concatenate(
        [src, jnp.zeros((pad,), jnp.int32)]).reshape(nrow, _IW)
    dst2d = jnp.concatenate(
        [dst, jnp.full((pad,), n, jnp.int32)]).reshape(nrow, _IW)

    ones_tbl = jnp.ones((n + 1, 8), jnp.float32)
    z32 = jnp.zeros((zsl, 32), jnp.float32)

    f1, f2, f3, f4 = W1.shape[1], W2.shape[1], W3.shape[1], W4.shape[1]
    f2p, f3p, f4p = max(f2, 8), max(f3, 8), max(f4, 8)

    def padw(w, r, c):
        return jnp.zeros((r, c), jnp.float32).at[:w.shape[0],
                                                 :w.shape[1]].set(w)

    W2p, W3p, W4p = padw(W2, f1, f2p), padw(W3, f2p, f3p), padw(W4, f3p, f4p)
    b2p = jnp.zeros((1, f2p), jnp.float32).at[0, :f2].set(b2)
    b3p = jnp.zeros((1, f3p), jnp.float32).at[0, :f3].set(b3)

    degp = _make_edge_pass(n, 8, nrow, rpt0, rpt1)(dst2d, dst2d, ones_tbl,
                                            z32[:, :8])
    dinv, g1 = _tc_pre(n, bsz, x.shape[1], f1)(degp[:, :, 0:1], x, W1)
    zb16 = jnp.zeros((zsl, f1), jnp.bfloat16)
    p1 = _make_edge_pass(n, f1, nrow, rpt0, rpt1, jnp.bfloat16)(
        src2d, dst2d, g1, zb16)
    g2 = _tc_mid(n, bsz, f1, f2p, False)(p1, g1, dinv, b1.reshape(1, f1),
                                         W2p)[0]
    p2 = _make_edge_pass(n, f2p, nrow, rpt0, rpt1)(src2d, dst2d, g2, z32[:, :f2p])
    g3 = _tc_mid(n, bsz, f2p, f3p, False)(p2, g2, dinv, b2p, W3p)[0]
    p3 = _make_edge_pass(n, f3p, nrow, rpt0, rpt1)(src2d, dst2d, g3, z32[:, :f3p])
    hp, g4 = _tc_mid(n, bsz, f3p, f4p, True)(p3, g3, dinv, b3p, W4p)
    p4 = _make_edge_pass(n, f4p, nrow, rpt0, rpt1)(src2d, dst2d, g4, z32[:, :f4p])
    y = _tc_post(n, bsz, f4)(p4, g4, dinv, b4.reshape(1, f4))
    return (hp[:, :f3], y)


# speedup vs baseline: 1.0305x; 1.0305x over previous
import functools

import jax
import jax.numpy as jnp
from jax import lax
from jax.experimental import pallas as pl
from jax.experimental.pallas import tpu as pltpu
from jax.experimental.pallas import tpu_sc as plsc

_NC = 2
_NS = 16
_IW = 128
_IB = 16
_SHARE0 = 0.92


def _make_edge_pass(n, f, nrow, rpt0, rpt1, dtype=jnp.float32):
    osl = -(-n // (_NS * 8)) * 8
    n_pad = max(osl * _NS, n + 8)
    zsl = n_pad // _NS
    last = n - (_NS - 1) * osl
    assert last > 0 and n_pad % _NS == 0 and zsl % 8 == 0
    mesh = plsc.VectorSubcoreMesh(
        core_axis_name="c", subcore_axis_name="s",
        num_cores=_NC, num_subcores=_NS)

    assert rpt0 % _IB == 0 and rpt1 % _IB == 0
    k_deep = 4 if (f > 8 and dtype == jnp.float32) else 16

    @functools.partial(
        pl.kernel,
        out_type=jax.ShapeDtypeStruct((_NC, n, f), dtype),
        mesh=mesh,
        compiler_params=pltpu.CompilerParams(use_tc_tiling_on_sc=False),
        scratch_types=[
            pltpu.VMEM((2, _IB, _IW), jnp.int32),
            pltpu.VMEM((2, _IB, _IW), jnp.int32),
            pltpu.VMEM((k_deep, _IW, f), dtype),
            pltpu.VMEM_SHARED((n_pad, f), dtype),
        ] + [pltpu.SemaphoreType.DMA] * (k_deep + 3),
    )
    def edge_pass(src_hbm, dst_hbm, tbl_hbm, zeros_hbm, out_hbm,
                  sblk, dblk, gbuf, acc, *sems):
        isems, gsems, ssem = sems[:2], sems[2:2 + k_deep], sems[2 + k_deep]
        cid = lax.axis_index("c")
        sid = lax.axis_index("s")

        pltpu.sync_copy(zeros_hbm, acc.at[pl.ds(sid * zsl, zsl)])
        plsc.subcore_barrier()

        def fire_idx(base, blk, slot):
            off = base + blk * _IB
            return (
                pltpu.async_copy(src_hbm.at[pl.ds(off, _IB)],
                                 sblk.at[slot, pl.ds(0, _IB)], isems[slot]),
                pltpu.async_copy(dst_hbm.at[pl.ds(off, _IB)],
                                 dblk.at[slot, pl.ds(0, _IB)], isems[slot]),
            )

        def run_rows(rpt_c, base):
            nblk = rpt_c // _IB
            if nblk == 0:
                return
            idx_d = fire_idx(base, 0, 0)
            for blk in range(nblk):
                slot = blk % 2
                idx_d[0].wait()
                idx_d[1].wait()
                if blk + 1 < nblk:
                    idx_d = fire_idx(base, blk + 1, 1 - slot)

                def group(g, carry, slot=slot):
                    g0 = g * k_deep
                    gds = [
                        pltpu.async_copy(tbl_hbm.at[sblk.at[slot, g0 + k]],
                                         gbuf.at[k], gsems[k])
                        for k in range(k_deep)
                    ]
                    sds = []
                    for k in range(k_deep):
                        gds[k].wait()
                        sds.append(
                            pltpu.async_copy(gbuf.at[k],
                                             acc.at[dblk.at[slot, g0 + k]],
                                             ssem, add=True))
                    for sd in sds:
                        sd.wait()
                    return carry

                lax.fori_loop(0, _IB // k_deep, group, 0)

        @pl.when(cid == 0)
        def _core0():
            run_rows(rpt0, sid * rpt0)

        @pl.when(cid == 1)
        def _core1():
            run_rows(rpt1, _NS * rpt0 + sid * rpt1)

        plsc.subcore_barrier()

        @pl.when(sid < _NS - 1)
        def _copy_main():
            pltpu.sync_copy(acc.at[pl.ds(sid * osl, osl)],
                            out_hbm.at[cid, pl.ds(sid * osl, osl)])

        @pl.when(sid == _NS - 1)
        def _copy_tail():
            pltpu.sync_copy(acc.at[pl.ds((_NS - 1) * osl, last)],
                            out_hbm.at[cid, pl.ds((_NS - 1) * osl, last)])

    return edge_pass


def _small_matmul(h, w_ref):
    f = w_ref.shape[0]
    acc = h[:, 0:1] * w_ref[0:1, :]
    for k in range(1, f):
        acc = acc + h[:, k:k + 1] * w_ref[k:k + 1, :]
    return acc


def _tc_pre(n, bsz, fin, fout):
    def body(degp_ref, x_ref, w_ref, dinv_ref, g_ref):
        deg = degp_ref[0] + degp_ref[1] + 1.0
        dinv = lax.rsqrt(deg)
        dinv_ref[...] = dinv
        g_ref[...] = (jnp.dot(x_ref[...], w_ref[...],
                              preferred_element_type=jnp.float32)
                      * dinv).astype(g_ref.dtype)

    return pl.pallas_call(
        body,
        grid=(n // bsz,),
        in_specs=[pl.BlockSpec((2, bsz, 1), lambda i: (0, i, 0)),
                  pl.BlockSpec((bsz, fin), lambda i: (i, 0)),
                  pl.BlockSpec((fin, fout), lambda i: (0, 0))],
        out_specs=[pl.BlockSpec((bsz, 1), lambda i: (i, 0)),
                   pl.BlockSpec((bsz, fout), lambda i: (i, 0))],
        out_shape=[jax.ShapeDtypeStruct((n, 1), jnp.float32),
                   jax.ShapeDtypeStruct((n, fout), jnp.bfloat16)])


def _tc_mid(n, bsz, f, f2, want_h):
    def body(p_ref, g_ref, dinv_ref, b_ref, w_ref, *out_refs):
        dinv = dinv_ref[...]
        s = (p_ref[0].astype(jnp.float32) + p_ref[1].astype(jnp.float32)
             + g_ref[...].astype(jnp.float32)) * dinv + b_ref[...]
        h = jnp.tanh(s)
        if want_h:
            out_refs[0][...] = h
        gn = _small_matmul(h, w_ref) if f <= 8 else jnp.dot(
            h, w_ref[...], preferred_element_type=jnp.float32)
        out_refs[-1][...] = gn * dinv

    out_specs = [pl.BlockSpec((bsz, f2), lambda i: (i, 0))]
    out_shape = [jax.ShapeDtypeStruct((n, f2), jnp.float32)]
    if want_h:
        out_specs.insert(0, pl.BlockSpec((bsz, f), lambda i: (i, 0)))
        out_shape.insert(0, jax.ShapeDtypeStruct((n, f), jnp.float32))
    return pl.pallas_call(
        body,
        grid=(n // bsz,),
        in_specs=[pl.BlockSpec((2, bsz, f), lambda i: (0, i, 0)),
                  pl.BlockSpec((bsz, f), lambda i: (i, 0)),
                  pl.BlockSpec((bsz, 1), lambda i: (i, 0)),
                  pl.BlockSpec((1, f), lambda i: (0, 0)),
                  pl.BlockSpec((f, f2), lambda i: (0, 0))],
        out_specs=out_specs,
        out_shape=out_shape)


def _tc_post(n, bsz, f):
    def body(p_ref, g_ref, dinv_ref, b_ref, y_ref):
        zp = (p_ref[0] + p_ref[1] + g_ref[...]) * dinv_ref[...]
        z = zp[:, :f] + b_ref[...]
        zm = z - jnp.max(z, axis=1, keepdims=True)
        y_ref[...] = zm - jnp.log(jnp.sum(jnp.exp(zm), axis=1, keepdims=True))

    fp = max(f, 8)
    return pl.pallas_call(
        body,
        grid=(n // bsz,),
        in_specs=[pl.BlockSpec((2, bsz, fp), lambda i: (0, i, 0)),
                  pl.BlockSpec((bsz, fp), lambda i: (i, 0)),
                  pl.BlockSpec((bsz, 1), lambda i: (i, 0)),
                  pl.BlockSpec((1, f), lambda i: (0, 0))],
        out_specs=pl.BlockSpec((bsz, f), lambda i: (i, 0)),
        out_shape=jax.ShapeDtypeStruct((n, f), jnp.float32))


def kernel(x, edge_index, W1, b1, W2, b2, W3, b3, W4, b4):
    n = x.shape[0]
    e = edge_index.shape[1]
    nw = _NC * _NS
    e_pad = -(-e // (nw * _IW * 8)) * (nw * _IW * 8)
    nrow = e_pad // _IW
    rpt_pair = nrow // _NS
    rpt0 = min(rpt_pair, max(_IB, int(round(rpt_pair * _SHARE0 / _IB)) * _IB))
    rpt1 = rpt_pair - rpt0
    osl = -(-n // (_NS * 8)) * 8
    zsl = max(osl * _NS, n + 8) // _NS
    bsz = 2000 if n % 2000 == 0 else n // 16

    src = edge_index[0].astype(jnp.int32)
    dst = edge_index[1].astype(jnp.int32)
    pad = e_pad - e
    src2d = jnp.concatenate(
        [src, jnp.zeros((pad,), jnp.int32)]).reshape(nrow, _IW)
    dst2d = jnp.concatenate(
        [dst, jnp.full((pad,), n, jnp.int32)]).reshape(nrow, _IW)

    ones_tbl = jnp.ones((n + 1, 8), jnp.float32)
    z32 = jnp.zeros((zsl, 32), jnp.float32)

    f1, f2, f3, f4 = W1.shape[1], W2.shape[1], W3.shape[1], W4.shape[1]
    f2p, f3p, f4p = max(f2, 8), max(f3, 8), max(f4, 8)

    def padw(w, r, c):
        return jnp.zeros((r, c), jnp.float32).at[:w.shape[0],
                                                 :w.shape[1]].set(w)

    W2p, W3p, W4p = padw(W2, f1, f2p), padw(W3, f2p, f3p), padw(W4, f3p, f4p)
    b2p = jnp.zeros((1, f2p), jnp.float32).at[0, :f2].set(b2)
    b3p = jnp.zeros((1, f3p), jnp.float32).at[0, :f3].set(b3)

    degp = _make_edge_pass(n, 8, nrow, rpt0, rpt1)(dst2d, dst2d, ones_tbl,
                                            z32[:, :8])
    dinv, g1 = _tc_pre(n, bsz, x.shape[1], f1)(degp[:, :, 0:1], x, W1)
    zb16 = jnp.zeros((zsl, f1), jnp.bfloat16)
    p1 = _make_edge_pass(n, f1, nrow, rpt0, rpt1, jnp.bfloat16)(
        src2d, dst2d, g1, zb16)
    g2 = _tc_mid(n, bsz, f1, f2p, False)(p1, g1, dinv, b1.reshape(1, f1),
                                         W2p)[0]
    p2 = _make_edge_pass(n, f2p, nrow, rpt0, rpt1)(src2d, dst2d, g2, z32[:, :f2p])
    g3 = _tc_mid(n, bsz, f2p, f3p, False)(p2, g2, dinv, b2p, W3p)[0]
    p3 = _make_edge_pass(n, f3p, nrow, rpt0, rpt1)(src2d, dst2d, g3, z32[:, :f3p])
    hp, g4 = _tc_mid(n, bsz, f3p, f4p, True)(p3, g3, dinv, b3p, W4p)
    p4 = _make_edge_pass(n, f4p, nrow, rpt0, rpt1)(src2d, dst2d, g4, z32[:, :f4p])
    y = _tc_post(n, bsz, f4)(p4, g4, dinv, b4.reshape(1, f4))
    return (hp[:, :f3], y)


# speedup vs baseline: 1.0317x; 1.0012x over previous
import functools

import jax
import jax.numpy as jnp
from jax import lax
from jax.experimental import pallas as pl
from jax.experimental.pallas import tpu as pltpu
from jax.experimental.pallas import tpu_sc as plsc

_NC = 2
_NS = 16
_IW = 128
_IB = 16
_SHARE0 = 0.92


def _make_edge_pass(n, f, nrow, rpt0, rpt1, dtype=jnp.float32):
    osl = -(-n // (_NS * 8)) * 8
    n_pad = max(osl * _NS, n + 8)
    zsl = n_pad // _NS
    last = n - (_NS - 1) * osl
    assert last > 0 and n_pad % _NS == 0 and zsl % 8 == 0
    mesh = plsc.VectorSubcoreMesh(
        core_axis_name="c", subcore_axis_name="s",
        num_cores=_NC, num_subcores=_NS)

    assert rpt0 % _IB == 0 and rpt1 % _IB == 0
    k_deep = 4 if (f > 8 and dtype == jnp.float32) else 8

    @functools.partial(
        pl.kernel,
        out_type=jax.ShapeDtypeStruct((_NC, n, f), dtype),
        mesh=mesh,
        compiler_params=pltpu.CompilerParams(use_tc_tiling_on_sc=False),
        scratch_types=[
            pltpu.VMEM((2, _IB, _IW), jnp.int32),
            pltpu.VMEM((2, _IB, _IW), jnp.int32),
            pltpu.VMEM((k_deep, _IW, f), dtype),
            pltpu.VMEM_SHARED((n_pad, f), dtype),
        ] + [pltpu.SemaphoreType.DMA] * (k_deep + 3),
    )
    def edge_pass(src_hbm, dst_hbm, tbl_hbm, zeros_hbm, out_hbm,
                  sblk, dblk, gbuf, acc, *sems):
        isems, gsems, ssem = sems[:2], sems[2:2 + k_deep], sems[2 + k_deep]
        cid = lax.axis_index("c")
        sid = lax.axis_index("s")

        pltpu.sync_copy(zeros_hbm, acc.at[pl.ds(sid * zsl, zsl)])
        plsc.subcore_barrier()

        def fire_idx(base, blk, slot):
            off = base + blk * _IB
            return (
                pltpu.async_copy(src_hbm.at[pl.ds(off, _IB)],
                                 sblk.at[slot, pl.ds(0, _IB)], isems[slot]),
                pltpu.async_copy(dst_hbm.at[pl.ds(off, _IB)],
                                 dblk.at[slot, pl.ds(0, _IB)], isems[slot]),
            )

        def run_rows(rpt_c, base):
            nblk = rpt_c // _IB
            if nblk == 0:
                return
            idx_d = fire_idx(base, 0, 0)
            for blk in range(nblk):
                slot = blk % 2
                idx_d[0].wait()
                idx_d[1].wait()
                if blk + 1 < nblk:
                    idx_d = fire_idx(base, blk + 1, 1 - slot)

                def group(g, carry, slot=slot):
                    g0 = g * k_deep
                    gds = [
                        pltpu.async_copy(tbl_hbm.at[sblk.at[slot, g0 + k]],
                                         gbuf.at[k], gsems[k])
                        for k in range(k_deep)
                    ]
                    sds = []
                    for k in range(k_deep):
                        gds[k].wait()
                        sds.append(
                            pltpu.async_copy(gbuf.at[k],
                                             acc.at[dblk.at[slot, g0 + k]],
                                             ssem, add=True))
                    for sd in sds:
                        sd.wait()
                    return carry

                lax.fori_loop(0, _IB // k_deep, group, 0)

        @pl.when(cid == 0)
        def _core0():
            run_rows(rpt0, sid * rpt0)

        @pl.when(cid == 1)
        def _core1():
            run_rows(rpt1, _NS * rpt0 + sid * rpt1)

        plsc.subcore_barrier()

        @pl.when(sid < _NS - 1)
        def _copy_main():
            pltpu.sync_copy(acc.at[pl.ds(sid * osl, osl)],
                            out_hbm.at[cid, pl.ds(sid * osl, osl)])

        @pl.when(sid == _NS - 1)
        def _copy_tail():
            pltpu.sync_copy(acc.at[pl.ds((_NS - 1) * osl, last)],
                            out_hbm.at[cid, pl.ds((_NS - 1) * osl, last)])

    return edge_pass


def _small_matmul(h, w_ref):
    f = w_ref.shape[0]
    acc = h[:, 0:1] * w_ref[0:1, :]
    for k in range(1, f):
        acc = acc + h[:, k:k + 1] * w_ref[k:k + 1, :]
    return acc


def _tc_pre(n, bsz, fin, fout):
    def body(degp_ref, x_ref, w_ref, dinv_ref, g_ref):
        deg = degp_ref[0] + degp_ref[1] + 1.0
        dinv = lax.rsqrt(deg)
        dinv_ref[...] = dinv
        g_ref[...] = (jnp.dot(x_ref[...], w_ref[...],
                              preferred_element_type=jnp.float32)
                      * dinv).astype(g_ref.dtype)

    return pl.pallas_call(
        body,
        grid=(n // bsz,),
        in_specs=[pl.BlockSpec((2, bsz, 1), lambda i: (0, i, 0)),
                  pl.BlockSpec((bsz, fin), lambda i: (i, 0)),
                  pl.BlockSpec((fin, fout), lambda i: (0, 0))],
        out_specs=[pl.BlockSpec((bsz, 1), lambda i: (i, 0)),
                   pl.BlockSpec((bsz, fout), lambda i: (i, 0))],
        out_shape=[jax.ShapeDtypeStruct((n, 1), jnp.float32),
                   jax.ShapeDtypeStruct((n, fout), jnp.bfloat16)])


def _tc_mid(n, bsz, f, f2, want_h):
    def body(p_ref, g_ref, dinv_ref, b_ref, w_ref, *out_refs):
        dinv = dinv_ref[...]
        s = (p_ref[0].astype(jnp.float32) + p_ref[1].astype(jnp.float32)
             + g_ref[...].astype(jnp.float32)) * dinv + b_ref[...]
        h = jnp.tanh(s)
        if want_h:
            out_refs[0][...] = h
        gn = _small_matmul(h, w_ref) if f <= 8 else jnp.dot(
            h, w_ref[...], preferred_element_type=jnp.float32)
        out_refs[-1][...] = gn * dinv

    out_specs = [pl.BlockSpec((bsz, f2), lambda i: (i, 0))]
    out_shape = [jax.ShapeDtypeStruct((n, f2), jnp.float32)]
    if want_h:
        out_specs.insert(0, pl.BlockSpec((bsz, f), lambda i: (i, 0)))
        out_shape.insert(0, jax.ShapeDtypeStruct((n, f), jnp.float32))
    return pl.pallas_call(
        body,
        grid=(n // bsz,),
        in_specs=[pl.BlockSpec((2, bsz, f), lambda i: (0, i, 0)),
                  pl.BlockSpec((bsz, f), lambda i: (i, 0)),
                  pl.BlockSpec((bsz, 1), lambda i: (i, 0)),
                  pl.BlockSpec((1, f), lambda i: (0, 0)),
                  pl.BlockSpec((f, f2), lambda i: (0, 0))],
        out_specs=out_specs,
        out_shape=out_shape)


def _tc_post(n, bsz, f):
    def body(p_ref, g_ref, dinv_ref, b_ref, y_ref):
        zp = (p_ref[0] + p_ref[1] + g_ref[...]) * dinv_ref[...]
        z = zp[:, :f] + b_ref[...]
        zm = z - jnp.max(z, axis=1, keepdims=True)
        y_ref[...] = zm - jnp.log(jnp.sum(jnp.exp(zm), axis=1, keepdims=True))

    fp = max(f, 8)
    return pl.pallas_call(
        body,
        grid=(n // bsz,),
        in_specs=[pl.BlockSpec((2, bsz, fp), lambda i: (0, i, 0)),
                  pl.BlockSpec((bsz, fp), lambda i: (i, 0)),
                  pl.BlockSpec((bsz, 1), lambda i: (i, 0)),
                  pl.BlockSpec((1, f), lambda i: (0, 0))],
        out_specs=pl.BlockSpec((bsz, f), lambda i: (i, 0)),
        out_shape=jax.ShapeDtypeStruct((n, f), jnp.float32))


def kernel(x, edge_index, W1, b1, W2, b2, W3, b3, W4, b4):
    n = x.shape[0]
    e = edge_index.shape[1]
    nw = _NC * _NS
    e_pad = -(-e // (nw * _IW * 8)) * (nw * _IW * 8)
    nrow = e_pad // _IW
    rpt_pair = nrow // _NS
    rpt0 = min(rpt_pair, max(_IB, int(round(rpt_pair * _SHARE0 / _IB)) * _IB))
    rpt1 = rpt_pair - rpt0
    osl = -(-n // (_NS * 8)) * 8
    zsl = max(osl * _NS, n + 8) // _NS
    bsz = 2000 if n % 2000 == 0 else n // 16

    src = edge_index[0].astype(jnp.int32)
    dst = edge_index[1].astype(jnp.int32)
    pad = e_pad - e
    src2d = jnp.concatenate(
        [src, jnp.zeros((pad,), jnp.int32)]).reshape(nrow, _IW)
    dst2d = jnp.concatenate(
        [dst, jnp.full((pad,), n, jnp.int32)]).reshape(nrow, _IW)

    ones_tbl = jnp.ones((n + 1, 8), jnp.float32)
    z32 = jnp.zeros((zsl, 32), jnp.float32)

    f1, f2, f3, f4 = W1.shape[1], W2.shape[1], W3.shape[1], W4.shape[1]
    f2p, f3p, f4p = max(f2, 8), max(f3, 8), max(f4, 8)

    def padw(w, r, c):
        return jnp.zeros((r, c), jnp.float32).at[:w.shape[0],
                                                 :w.shape[1]].set(w)

    W2p, W3p, W4p = padw(W2, f1, f2p), padw(W3, f2p, f3p), padw(W4, f3p, f4p)
    b2p = jnp.zeros((1, f2p), jnp.float32).at[0, :f2].set(b2)
    b3p = jnp.zeros((1, f3p), jnp.float32).at[0, :f3].set(b3)

    degp = _make_edge_pass(n, 8, nrow, rpt0, rpt1)(dst2d, dst2d, ones_tbl,
                                            z32[:, :8])
    dinv, g1 = _tc_pre(n, bsz, x.shape[1], f1)(degp[:, :, 0:1], x, W1)
    zb16 = jnp.zeros((zsl, f1), jnp.bfloat16)
    p1 = _make_edge_pass(n, f1, nrow, rpt0, rpt1, jnp.bfloat16)(
        src2d, dst2d, g1, zb16)
    g2 = _tc_mid(n, bsz, f1, f2p, False)(p1, g1, dinv, b1.reshape(1, f1),
                                         W2p)[0]
    p2 = _make_edge_pass(n, f2p, nrow, rpt0, rpt1)(src2d, dst2d, g2, z32[:, :f2p])
    g3 = _tc_mid(n, bsz, f2p, f3p, False)(p2, g2, dinv, b2p, W3p)[0]
    p3 = _make_edge_pass(n, f3p, nrow, rpt0, rpt1)(src2d, dst2d, g3, z32[:, :f3p])
    hp, g4 = _tc_mid(n, bsz, f3p, f4p, True)(p3, g3, dinv, b3p, W4p)
    p4 = _make_edge_pass(n, f4p, nrow, rpt0, rpt1)(src2d, dst2d, g4, z32[:, :f4p])
    y = _tc_post(n, bsz, f4)(p4, g4, dinv, b4.reshape(1, f4))
    return (hp[:, :f3], y)
